# baseline (speedup 1.0000x reference)
import jax
import jax.numpy as jnp
from jax.experimental import pallas as pl
from jax.experimental.pallas import tpu as pltpu

B, S, D = 4, 8192, 2048
SB = 512
NSB = S // SB


def _body(x_ref, o_ref):
    s = pl.program_id(1)
    part = jnp.sum(x_ref[...], axis=1, keepdims=True) * jnp.float32(1.0 / S)

    @pl.when(s == 0)
    def _():
        o_ref[...] = part

    @pl.when(s > 0)
    def _():
        o_ref[...] += part


def kernel(embeds):
    return pl.pallas_call(
        _body,
        grid=(B, NSB),
        in_specs=[pl.BlockSpec((1, SB, D), lambda b, s: (b, s, 0))],
        out_specs=pl.BlockSpec((1, 1, D), lambda b, s: (b, 0, 0)),
        out_shape=jax.ShapeDtypeStruct((B, 1, D), jnp.float32),
        compiler_params=pltpu.CompilerParams(
            dimension_semantics=("parallel", "arbitrary"),
        ),
    )(embeds)


# speedup vs baseline: 1.0634x; 1.0634x over previous
import jax
import jax.numpy as jnp
from jax.experimental import pallas as pl
from jax.experimental.pallas import tpu as pltpu

B, S, D = 4, 8192, 2048
SB = 1024
NSB = S // SB


def _body(x_ref, o_ref):
    s = pl.program_id(1)
    part = jnp.sum(x_ref[...], axis=1, keepdims=True) * jnp.float32(1.0 / S)

    @pl.when(s == 0)
    def _():
        o_ref[...] = part

    @pl.when(s > 0)
    def _():
        o_ref[...] += part


def kernel(embeds):
    return pl.pallas_call(
        _body,
        grid=(B, NSB),
        in_specs=[pl.BlockSpec((1, SB, D), lambda b, s: (b, s, 0))],
        out_specs=pl.BlockSpec((1, 1, D), lambda b, s: (b, 0, 0)),
        out_shape=jax.ShapeDtypeStruct((B, 1, D), jnp.float32),
        compiler_params=pltpu.CompilerParams(
            dimension_semantics=("parallel", "arbitrary"),
        ),
    )(embeds)
